# baseline (speedup 1.0000x reference)
import jax
import jax.numpy as jnp
from jax import lax
from jax.experimental import pallas as pl
from jax.experimental.pallas import tpu as pltpu
from jax.experimental.pallas import tpu_sc as plsc

_B, _S, _D = 4096, 200, 64
_TOTAL = _B * _S
_NC, _NS = 2, 16
_NW = _NC * _NS
_PER_W = _TOTAL // _NW
_G = 128
_NG = _PER_W // _G


def _emb_body(idx_hbm, emb_hbm, out_hbm, idx_v, rows_v, sem):
    wid = lax.axis_index("s") * _NC + lax.axis_index("c")
    pltpu.sync_copy(idx_hbm.at[wid], idx_v)
    base = wid * _PER_W

    def step(j, carry):
        pltpu.async_copy(emb_hbm.at[idx_v.at[j]], rows_v, sem).wait()
        pltpu.sync_copy(rows_v, out_hbm.at[pl.ds(base + j * _G, _G)])
        return carry

    lax.fori_loop(0, _NG, step, 0)


def kernel(x, emb):
    idx = x.astype(jnp.int32).reshape(_NW, _NG, _G)
    run = pl.kernel(
        _emb_body,
        out_type=jax.ShapeDtypeStruct((_TOTAL, _D), jnp.float32),
        mesh=plsc.VectorSubcoreMesh(core_axis_name="c", subcore_axis_name="s"),
        compiler_params=pltpu.CompilerParams(use_tc_tiling_on_sc=False),
        scratch_types=[
            pltpu.VMEM((_NG, _G), jnp.int32),
            pltpu.VMEM((_G, _D), jnp.float32),
            pltpu.SemaphoreType.DMA,
        ],
    )
    out = run(idx, emb)
    return out.reshape(_B, _S, _D)


# speedup vs baseline: 1.0870x; 1.0870x over previous
import jax
import jax.numpy as jnp
from jax import lax
from jax.experimental import pallas as pl
from jax.experimental.pallas import tpu as pltpu
from jax.experimental.pallas import tpu_sc as plsc

_B, _S, _D = 4096, 200, 64
_TOTAL = _B * _S
_NC, _NS = 2, 16
_NW = _NC * _NS
_PER_W = _TOTAL // _NW
_G = 512
_NG = _PER_W // _G


def _emb_body(idx_hbm, emb_hbm, out_hbm, idx_v, rows_v, sem):
    wid = lax.axis_index("s") * _NC + lax.axis_index("c")
    pltpu.sync_copy(idx_hbm.at[wid], idx_v)
    base = wid * _PER_W

    def step(j, carry):
        pltpu.async_copy(emb_hbm.at[idx_v.at[j]], rows_v, sem).wait()
        pltpu.sync_copy(rows_v, out_hbm.at[pl.ds(base + j * _G, _G)])
        return carry

    lax.fori_loop(0, _NG, step, 0)


def kernel(x, emb):
    idx = x.astype(jnp.int32).reshape(_NW, _NG, _G)
    run = pl.kernel(
        _emb_body,
        out_type=jax.ShapeDtypeStruct((_TOTAL, _D), jnp.float32),
        mesh=plsc.VectorSubcoreMesh(core_axis_name="c", subcore_axis_name="s"),
        compiler_params=pltpu.CompilerParams(use_tc_tiling_on_sc=False),
        scratch_types=[
            pltpu.VMEM((_NG, _G), jnp.int32),
            pltpu.VMEM((_G, _D), jnp.float32),
            pltpu.SemaphoreType.DMA,
        ],
    )
    out = run(idx, emb)
    return out.reshape(_B, _S, _D)


# speedup vs baseline: 1.1127x; 1.0236x over previous
import jax
import jax.numpy as jnp
from jax import lax
from jax.experimental import pallas as pl
from jax.experimental.pallas import tpu as pltpu
from jax.experimental.pallas import tpu_sc as plsc

_B, _S, _D = 4096, 200, 64
_TOTAL = _B * _S
_NC, _NS = 2, 16
_NW = _NC * _NS
_PER_W = _TOTAL // _NW
_G = 128
_NG = _PER_W // _G
_NBUF = 8
_T = _NG // _NBUF


def _gather(emb_hbm, idx_v, rows_v, j, b, sem):
    return pltpu.async_copy(emb_hbm.at[idx_v.at[j]], rows_v.at[b], sem)


def _gather_wait(emb_hbm, rows_v, b, sem):
    pltpu.make_async_copy(emb_hbm.at[pl.ds(0, _G)], rows_v.at[b], sem).wait()


def _store(out_hbm, rows_v, base, j, b, sem):
    return pltpu.async_copy(rows_v.at[b], out_hbm.at[pl.ds(base + j * _G, _G)],
                            sem)


def _store_wait(out_hbm, rows_v, b, sem):
    pltpu.make_async_copy(rows_v.at[b], out_hbm.at[pl.ds(0, _G)], sem).wait()


def _emb_body(idx_hbm, emb_hbm, out_hbm, idx_v, rows_v, *sems):
    gsems, ssems = sems[:_NBUF], sems[_NBUF:]
    wid = lax.axis_index("s") * _NC + lax.axis_index("c")
    pltpu.sync_copy(idx_hbm.at[wid], idx_v)
    base = wid * _PER_W

    for b in range(_NBUF):
        _gather(emb_hbm, idx_v, rows_v, b, b, gsems[b])

    def body(t, carry):
        j0 = t * _NBUF
        for b in range(_NBUF):
            _gather_wait(emb_hbm, rows_v, b, gsems[b])
            _store(out_hbm, rows_v, base, j0 + b, b, ssems[b])

        @pl.when(t + 1 < _T)
        def _():
            for b in range(_NBUF):
                _store_wait(out_hbm, rows_v, b, ssems[b])
                _gather(emb_hbm, idx_v, rows_v, j0 + _NBUF + b, b, gsems[b])

        return carry

    lax.fori_loop(0, _T, body, 0)
    for b in range(_NBUF):
        _store_wait(out_hbm, rows_v, b, ssems[b])


def kernel(x, emb):
    idx = x.astype(jnp.int32).reshape(_NW, _NG, _G)
    run = pl.kernel(
        _emb_body,
        out_type=jax.ShapeDtypeStruct((_TOTAL, _D), jnp.float32),
        mesh=plsc.VectorSubcoreMesh(core_axis_name="c", subcore_axis_name="s"),
        compiler_params=pltpu.CompilerParams(use_tc_tiling_on_sc=False),
        scratch_types=(
            [pltpu.VMEM((_NG, _G), jnp.int32),
             pltpu.VMEM((_NBUF, _G, _D), jnp.float32)]
            + [pltpu.SemaphoreType.DMA] * (2 * _NBUF)
        ),
    )
    out = run(idx, emb)
    return out.reshape(_B, _S, _D)
